# Initial kernel scaffold; baseline (speedup 1.0000x reference)
import jax
import jax.numpy as jnp
from jax.experimental import pallas as pl


def kernel(input, indices):
    raise NotImplementedError("write your pallas kernel here")


# speedup vs baseline: 21.4541x; 21.4541x over previous
import functools

import jax
import jax.numpy as jnp
from jax import lax
from jax.experimental import pallas as pl
from jax.experimental.pallas import tpu as pltpu
from jax.experimental.pallas import tpu_sc as plsc

B, H, W, C = 4, 112, 112, 192
HW = H * W
HO, WO = 2 * H, 2 * W
P = HO * WO
NCORE, NSUB = 2, 16
NWORK = NCORE * NSUB
MPERW = C // NWORK
BHW = 1792
BP = 1792
INV_C = 1.0 / C


def _tin_body(x_ref, i_ref, vt_ref, pt_ref):
    v = x_ref[0]
    idx = i_ref[0]
    p0 = (idx.astype(jnp.float32) * INV_C).astype(jnp.int32)
    r = idx - p0 * C
    p = p0 + (r >= C).astype(jnp.int32) - (r < 0).astype(jnp.int32)
    vt_ref[0] = v.T
    pt_ref[0] = p.T


def _transpose_in(inp3, idx3):
    return pl.pallas_call(
        _tin_body,
        grid=(B, HW // BHW),
        in_specs=[
            pl.BlockSpec((1, BHW, C), lambda b, i: (b, i, 0)),
            pl.BlockSpec((1, BHW, C), lambda b, i: (b, i, 0)),
        ],
        out_specs=[
            pl.BlockSpec((1, C, BHW), lambda b, i: (b, 0, i)),
            pl.BlockSpec((1, C, BHW), lambda b, i: (b, 0, i)),
        ],
        out_shape=[
            jax.ShapeDtypeStruct((B, C, HW), jnp.float32),
            jax.ShapeDtypeStruct((B, C, HW), jnp.int32),
        ],
    )(inp3, idx3)


def _tout_body(t_ref, o_ref):
    o_ref[0] = t_ref[0].T


def _transpose_out(out_t):
    return pl.pallas_call(
        _tout_body,
        grid=(B, P // BP),
        in_specs=[pl.BlockSpec((1, C, BP), lambda b, i: (b, 0, i))],
        out_specs=pl.BlockSpec((1, BP, C), lambda b, i: (b, i, 0)),
        out_shape=jax.ShapeDtypeStruct((B, P, C), jnp.float32),
    )(out_t)


_mesh = plsc.VectorSubcoreMesh(core_axis_name="c", subcore_axis_name="s")


@functools.partial(
    pl.kernel,
    out_type=jax.ShapeDtypeStruct((B, C, P), jnp.float32),
    mesh=_mesh,
    scratch_types=[
        pltpu.VMEM((P,), jnp.float32),
        pltpu.VMEM((HW,), jnp.float32),
        pltpu.VMEM((HW,), jnp.int32),
    ],
    compiler_params=pltpu.CompilerParams(needs_layout_passes=False),
)
def _scatter(vals_hbm, p_hbm, out_hbm, canvas, vbuf, pbuf):
    w = lax.axis_index("s") * NCORE + lax.axis_index("c")

    def _zero(i, carry):
        canvas[pl.ds(i * 16, 16)] = jnp.zeros((16,), jnp.float32)
        return carry

    lax.fori_loop(0, P // 16, _zero, 0)

    def _batch(b, carry):
        def _slot(m, carry2):
            c = w + NWORK * m
            pltpu.sync_copy(vals_hbm.at[b, c], vbuf)
            pltpu.sync_copy(p_hbm.at[b, c], pbuf)

            def _scat(j, carry3):
                idxv = pbuf[pl.ds(j * 16, 16)]
                valv = vbuf[pl.ds(j * 16, 16)]
                plsc.addupdate_scatter(canvas, [idxv], valv)
                return carry3

            lax.fori_loop(0, HW // 16, _scat, 0)
            pltpu.sync_copy(canvas, out_hbm.at[b, c])

            def _clear(j, carry3):
                idxv = pbuf[pl.ds(j * 16, 16)]
                plsc.store_scatter(canvas, [idxv], jnp.zeros((16,), jnp.float32))
                return carry3

            lax.fori_loop(0, HW // 16, _clear, 0)
            return carry2

        return lax.fori_loop(0, MPERW, _slot, carry)

    lax.fori_loop(0, B, _batch, 0)


def kernel(input, indices):
    inp3 = input.reshape(B, HW, C)
    idx3 = indices.astype(jnp.int32).reshape(B, HW, C)
    vt, pt = _transpose_in(inp3, idx3)
    out_t = _scatter(vt, pt)
    out = _transpose_out(out_t)
    return out.reshape(B, HO, WO, C)


# speedup vs baseline: 39.9824x; 1.8636x over previous
import functools

import jax
import jax.numpy as jnp
from jax import lax
from jax.experimental import pallas as pl
from jax.experimental.pallas import tpu as pltpu
from jax.experimental.pallas import tpu_sc as plsc

B, H, W, C = 4, 112, 112, 192
HW = H * W
HO, WO = 2 * H, 2 * W
P = HO * WO
NCORE, NSUB = 2, 16
NWORK = NCORE * NSUB
MPERW = C // NWORK
NPLANES = B * MPERW
HB = 16
YB = 16
INV_C = 1.0 / C


def _group_body(x_ref, i_ref, vt_ref, pt_ref):
    v = x_ref[0]
    idx = i_ref[0]
    p0 = (idx.astype(jnp.float32) * INV_C).astype(jnp.int32)
    r = idx - p0 * C
    p = p0 + (r >= C).astype(jnp.int32) - (r < 0).astype(jnp.int32)
    vt_ref[0] = jnp.swapaxes(v, 0, 1)
    pt_ref[0] = jnp.swapaxes(p, 0, 1)


def _group_channels(x, ii):
    return pl.pallas_call(
        _group_body,
        grid=(B, H // HB),
        in_specs=[
            pl.BlockSpec((1, HB, C, W), lambda b, i: (b, i, 0, 0)),
            pl.BlockSpec((1, HB, C, W), lambda b, i: (b, i, 0, 0)),
        ],
        out_specs=[
            pl.BlockSpec((1, C, HB, W), lambda b, i: (b, 0, i, 0)),
            pl.BlockSpec((1, C, HB, W), lambda b, i: (b, 0, i, 0)),
        ],
        out_shape=[
            jax.ShapeDtypeStruct((B, C, H, W), jnp.float32),
            jax.ShapeDtypeStruct((B, C, H, W), jnp.int32),
        ],
    )(x, ii)


def _relayout_body(t_ref, o_ref):
    o_ref[0] = jnp.swapaxes(t_ref[0], 0, 1)


def _relayout_out(out_t):
    return pl.pallas_call(
        _relayout_body,
        grid=(B, HO // YB),
        in_specs=[pl.BlockSpec((1, C, YB, WO), lambda b, y: (b, 0, y, 0))],
        out_specs=pl.BlockSpec((1, YB, C, WO), lambda b, y: (b, y, 0, 0)),
        out_shape=jax.ShapeDtypeStruct((B, HO, C, WO), jnp.float32),
    )(out_t)


_mesh = plsc.VectorSubcoreMesh(core_axis_name="c", subcore_axis_name="s")


@functools.partial(
    pl.kernel,
    out_type=jax.ShapeDtypeStruct((B, C, P), jnp.float32),
    mesh=_mesh,
    scratch_types=[
        pltpu.VMEM((P,), jnp.float32),
        pltpu.VMEM((2 * HW,), jnp.float32),
        pltpu.VMEM((2 * HW,), jnp.int32),
        pltpu.SemaphoreType.DMA((2,)),
    ],
    compiler_params=pltpu.CompilerParams(needs_layout_passes=False),
)
def _scatter(vals_hbm, p_hbm, out_hbm, canvas, vbuf, pbuf, lsem):
    w = lax.axis_index("s") * NCORE + lax.axis_index("c")
    zero16 = jnp.zeros((16,), jnp.float32)

    def _load(b, m, sel):
        c = w + NWORK * m
        pltpu.async_copy(vals_hbm.at[b, c], vbuf.at[pl.ds(sel * HW, HW)], lsem.at[sel])
        pltpu.async_copy(p_hbm.at[b, c], pbuf.at[pl.ds(sel * HW, HW)], lsem.at[sel])

    _load(jnp.int32(0), jnp.int32(0), jnp.int32(0))

    def _zero(i, carry):
        canvas[pl.ds(i * 16, 16)] = zero16
        return carry

    lax.fori_loop(0, P // 16, _zero, 0)

    def _plane(k, carry):
        b, m = carry
        sel = jnp.bitwise_and(k, 1)
        m2 = m + 1
        wrap = (m2 == MPERW).astype(jnp.int32)
        nm = jnp.where(m2 == MPERW, 0, m2)
        nb = b + wrap

        @pl.when(k < NPLANES - 1)
        def _():
            _load(nb, nm, 1 - sel)

        c = w + NWORK * m
        base = sel * HW
        pltpu.make_async_copy(
            vals_hbm.at[b, c], vbuf.at[pl.ds(base, HW)], lsem.at[sel]).wait()
        pltpu.make_async_copy(
            p_hbm.at[b, c], pbuf.at[pl.ds(base, HW)], lsem.at[sel]).wait()

        def _scat(j, carry3):
            o = base + j * 64
            for u in range(4):
                idxv = pbuf[pl.ds(o + u * 16, 16)]
                valv = vbuf[pl.ds(o + u * 16, 16)]
                plsc.addupdate_scatter(canvas, [idxv], valv)
            return carry3

        lax.fori_loop(0, HW // 64, _scat, 0)
        pltpu.sync_copy(canvas, out_hbm.at[b, c])

        def _clear(j, carry3):
            o = base + j * 64
            for u in range(4):
                idxv = pbuf[pl.ds(o + u * 16, 16)]
                plsc.store_scatter(canvas, [idxv], zero16)
            return carry3

        lax.fori_loop(0, HW // 64, _clear, 0)
        return (nb, nm)

    lax.fori_loop(0, NPLANES, _plane, (jnp.int32(0), jnp.int32(0)))


def kernel(input, indices):
    x = jnp.transpose(input, (0, 1, 3, 2))
    ii = jnp.transpose(indices.astype(jnp.int32), (0, 1, 3, 2))
    vt, pt = _group_channels(x, ii)
    out_t = _scatter(vt.reshape(B, C, HW), pt.reshape(B, C, HW))
    out_phys = _relayout_out(out_t.reshape(B, C, HO, WO))
    return jnp.transpose(out_phys, (0, 1, 3, 2))


# speedup vs baseline: 51.0649x; 1.2772x over previous
import functools

import jax
import jax.numpy as jnp
from jax import lax
from jax.experimental import pallas as pl
from jax.experimental.pallas import tpu as pltpu
from jax.experimental.pallas import tpu_sc as plsc

B, H, W, C = 4, 112, 112, 192
HW = H * W
HO, WO = 2 * H, 2 * W
NCORE, NSUB = 2, 16
NWORK = NCORE * NSUB
MPERW = C // NWORK
NPLANES = B * MPERW
HB = 16
INV_WC = 1.0 / (WO * C)
INV_C = 1.0 / C


def _group_body(x_ref, i_ref, vt_ref, pt_ref):
    v = x_ref[0]
    idx = i_ref[0]
    y0 = (idx.astype(jnp.float32) * INV_WC).astype(jnp.int32)
    r = idx - y0 * (WO * C)
    fix = (r >= WO * C).astype(jnp.int32) - (r < 0).astype(jnp.int32)
    y = y0 + fix
    r = r - fix * (WO * C)
    x0 = (r.astype(jnp.float32) * INV_C).astype(jnp.int32)
    r2 = r - x0 * C
    x = x0 + (r2 >= C).astype(jnp.int32) - (r2 < 0).astype(jnp.int32)
    packed = jnp.left_shift(y, 8) + x
    vt_ref[0] = jnp.swapaxes(v, 0, 1)
    pt_ref[0] = jnp.swapaxes(packed, 0, 1)


def _group_channels(x, ii):
    return pl.pallas_call(
        _group_body,
        grid=(B, H // HB),
        in_specs=[
            pl.BlockSpec((1, HB, C, W), lambda b, i: (b, i, 0, 0)),
            pl.BlockSpec((1, HB, C, W), lambda b, i: (b, i, 0, 0)),
        ],
        out_specs=[
            pl.BlockSpec((1, C, HB, W), lambda b, i: (b, 0, i, 0)),
            pl.BlockSpec((1, C, HB, W), lambda b, i: (b, 0, i, 0)),
        ],
        out_shape=[
            jax.ShapeDtypeStruct((B, C, H, W), jnp.float32),
            jax.ShapeDtypeStruct((B, C, H, W), jnp.int32),
        ],
    )(x, ii)


_mesh = plsc.VectorSubcoreMesh(core_axis_name="c", subcore_axis_name="s")


@functools.partial(
    pl.kernel,
    out_type=jax.ShapeDtypeStruct((B, HO, C, WO), jnp.float32),
    mesh=_mesh,
    scratch_types=[
        pltpu.VMEM((HO, WO), jnp.float32),
        pltpu.VMEM((2 * HW,), jnp.float32),
        pltpu.VMEM((2 * HW,), jnp.int32),
        pltpu.SemaphoreType.DMA((2,)),
    ],
    compiler_params=pltpu.CompilerParams(needs_layout_passes=False),
)
def _scatter(vals_hbm, p_hbm, out_hbm, canvas, vbuf, pbuf, lsem):
    w = lax.axis_index("s") * NCORE + lax.axis_index("c")
    zero16 = jnp.zeros((16,), jnp.float32)
    mask8 = jnp.full((16,), 255, jnp.int32)

    def _load(b, m, sel):
        c = w + NWORK * m
        pltpu.async_copy(vals_hbm.at[b, c], vbuf.at[pl.ds(sel * HW, HW)], lsem.at[sel])
        pltpu.async_copy(p_hbm.at[b, c], pbuf.at[pl.ds(sel * HW, HW)], lsem.at[sel])

    _load(jnp.int32(0), jnp.int32(0), jnp.int32(0))

    def _zero(y, carry):
        def _zrow(j, carry2):
            canvas[y, pl.ds(j * 16, 16)] = zero16
            return carry2
        return lax.fori_loop(0, WO // 16, _zrow, carry)

    lax.fori_loop(0, HO, _zero, 0)

    def _plane(k, carry):
        b, m = carry
        sel = jnp.bitwise_and(k, 1)
        m2 = m + 1
        wrap = (m2 == MPERW).astype(jnp.int32)
        nm = jnp.where(m2 == MPERW, 0, m2)
        nb = b + wrap

        @pl.when(k < NPLANES - 1)
        def _():
            _load(nb, nm, 1 - sel)

        c = w + NWORK * m
        base = sel * HW
        pltpu.make_async_copy(
            vals_hbm.at[b, c], vbuf.at[pl.ds(base, HW)], lsem.at[sel]).wait()
        pltpu.make_async_copy(
            p_hbm.at[b, c], pbuf.at[pl.ds(base, HW)], lsem.at[sel]).wait()

        def _scat(j, carry3):
            o = base + j * 64
            for u in range(4):
                pv = pbuf[pl.ds(o + u * 16, 16)]
                yv = jnp.right_shift(pv, 8)
                xv = jnp.bitwise_and(pv, mask8)
                valv = vbuf[pl.ds(o + u * 16, 16)]
                plsc.addupdate_scatter(canvas, [yv, xv], valv)
            return carry3

        lax.fori_loop(0, HW // 64, _scat, 0)
        pltpu.sync_copy(canvas, out_hbm.at[b, :, c, :])

        def _clear(j, carry3):
            o = base + j * 64
            for u in range(4):
                pv = pbuf[pl.ds(o + u * 16, 16)]
                yv = jnp.right_shift(pv, 8)
                xv = jnp.bitwise_and(pv, mask8)
                plsc.store_scatter(canvas, [yv, xv], zero16)
            return carry3

        lax.fori_loop(0, HW // 64, _clear, 0)
        return (nb, nm)

    lax.fori_loop(0, NPLANES, _plane, (jnp.int32(0), jnp.int32(0)))


def kernel(input, indices):
    x = jnp.transpose(input, (0, 1, 3, 2))
    ii = jnp.transpose(indices.astype(jnp.int32), (0, 1, 3, 2))
    vt, pt = _group_channels(x, ii)
    out_phys = _scatter(vt.reshape(B, C, HW), pt.reshape(B, C, HW))
    return jnp.transpose(out_phys, (0, 1, 3, 2))


# speedup vs baseline: 55.6225x; 1.0893x over previous
import functools

import jax
import jax.numpy as jnp
from jax import lax
from jax.experimental import pallas as pl
from jax.experimental.pallas import tpu as pltpu
from jax.experimental.pallas import tpu_sc as plsc

B, H, W, C = 4, 112, 112, 192
HW = H * W
HO, WO = 2 * H, 2 * W
P = HO * WO
NCORE, NSUB = 2, 16
NWORK = NCORE * NSUB
MPERW = C // NWORK
HB = 16
YB = 16
INV_C = 1.0 / C


def _group_body(x_ref, i_ref, vt_ref, pt_ref):
    v = x_ref[0]
    idx = i_ref[0]
    p0 = (idx.astype(jnp.float32) * INV_C).astype(jnp.int32)
    r = idx - p0 * C
    p = p0 + (r >= C).astype(jnp.int32) - (r < 0).astype(jnp.int32)
    vt_ref[...] = jnp.swapaxes(v, 0, 1)
    pt_ref[...] = jnp.swapaxes(p, 0, 1)


def _group_channels(x, ii, b):
    return pl.pallas_call(
        _group_body,
        grid=(H // HB,),
        in_specs=[
            pl.BlockSpec((1, HB, C, W), lambda i: (b, i, 0, 0)),
            pl.BlockSpec((1, HB, C, W), lambda i: (b, i, 0, 0)),
        ],
        out_specs=[
            pl.BlockSpec((C, HB, W), lambda i: (0, i, 0)),
            pl.BlockSpec((C, HB, W), lambda i: (0, i, 0)),
        ],
        out_shape=[
            jax.ShapeDtypeStruct((C, H, W), jnp.float32),
            jax.ShapeDtypeStruct((C, H, W), jnp.int32),
        ],
    )(x, ii)


def _relayout_first_body(t_ref, o_ref):
    o_ref[0] = jnp.swapaxes(t_ref[...], 0, 1)


def _relayout_accum_body(prev_ref, t_ref, o_ref):
    del prev_ref
    o_ref[0] = jnp.swapaxes(t_ref[...], 0, 1)


def _relayout_out(out_t, b, prev):
    in_specs = [pl.BlockSpec((C, YB, WO), lambda y: (0, y, 0))]
    if prev is None:
        return pl.pallas_call(
            _relayout_first_body,
            grid=(HO // YB,),
            in_specs=in_specs,
            out_specs=pl.BlockSpec((1, YB, C, WO), lambda y: (b, y, 0, 0)),
            out_shape=jax.ShapeDtypeStruct((B, HO, C, WO), jnp.float32),
        )(out_t)
    return pl.pallas_call(
        _relayout_accum_body,
        grid=(HO // YB,),
        in_specs=[pl.BlockSpec(memory_space=pl.ANY)] + in_specs,
        out_specs=pl.BlockSpec((1, YB, C, WO), lambda y: (b, y, 0, 0)),
        out_shape=jax.ShapeDtypeStruct((B, HO, C, WO), jnp.float32),
        input_output_aliases={0: 0},
    )(prev, out_t)


_mesh = plsc.VectorSubcoreMesh(core_axis_name="c", subcore_axis_name="s")


@functools.partial(
    pl.kernel,
    out_type=jax.ShapeDtypeStruct((C, P), jnp.float32),
    mesh=_mesh,
    scratch_types=[
        pltpu.VMEM((P,), jnp.float32),
        pltpu.VMEM((2 * HW,), jnp.float32),
        pltpu.VMEM((2 * HW,), jnp.int32),
        pltpu.SemaphoreType.DMA((2,)),
    ],
    compiler_params=pltpu.CompilerParams(needs_layout_passes=False),
)
def _scatter(vals_hbm, p_hbm, out_hbm, canvas, vbuf, pbuf, lsem):
    w = lax.axis_index("s") * NCORE + lax.axis_index("c")
    zero16 = jnp.zeros((16,), jnp.float32)

    def _load(m, sel):
        c = w + NWORK * m
        pltpu.async_copy(vals_hbm.at[c], vbuf.at[pl.ds(sel * HW, HW)], lsem.at[sel])
        pltpu.async_copy(p_hbm.at[c], pbuf.at[pl.ds(sel * HW, HW)], lsem.at[sel])

    _load(jnp.int32(0), jnp.int32(0))

    def _zero(i, carry):
        canvas[pl.ds(i * 16, 16)] = zero16
        return carry

    lax.fori_loop(0, P // 16, _zero, 0)

    def _plane(m, carry):
        sel = jnp.bitwise_and(m, 1)

        @pl.when(m < MPERW - 1)
        def _():
            _load(m + 1, 1 - sel)

        c = w + NWORK * m
        base = sel * HW
        pltpu.make_async_copy(
            vals_hbm.at[c], vbuf.at[pl.ds(base, HW)], lsem.at[sel]).wait()
        pltpu.make_async_copy(
            p_hbm.at[c], pbuf.at[pl.ds(base, HW)], lsem.at[sel]).wait()

        def _scat(j, carry3):
            o = base + j * 64
            for u in range(4):
                idxv = pbuf[pl.ds(o + u * 16, 16)]
                valv = vbuf[pl.ds(o + u * 16, 16)]
                plsc.addupdate_scatter(canvas, [idxv], valv)
            return carry3

        lax.fori_loop(0, HW // 64, _scat, 0)
        pltpu.sync_copy(canvas, out_hbm.at[c])

        def _clear(j, carry3):
            o = base + j * 64
            for u in range(4):
                idxv = pbuf[pl.ds(o + u * 16, 16)]
                plsc.store_scatter(canvas, [idxv], zero16)
            return carry3

        lax.fori_loop(0, HW // 64, _clear, 0)
        return carry

    lax.fori_loop(0, MPERW, _plane, 0)


def kernel(input, indices):
    x = jnp.transpose(input, (0, 1, 3, 2))
    ii = jnp.transpose(indices.astype(jnp.int32), (0, 1, 3, 2))
    out = None
    for b in range(B):
        vt, pt = _group_channels(x, ii, b)
        out_t = _scatter(vt.reshape(C, HW), pt.reshape(C, HW))
        out = _relayout_out(out_t.reshape(C, HO, WO), b, out)
    return jnp.transpose(out, (0, 1, 3, 2))
